# baseline (speedup 1.0000x reference)
import functools

import jax
import jax.numpy as jnp
from jax import lax
from jax.experimental import pallas as pl
from jax.experimental.pallas import tpu as pltpu
from jax.experimental.pallas import tpu_sc as plsc

_NUM_CLASSES = 1000
_HIDDEN = 128
_BATCH = 16384

_INFO = plsc.get_sparse_core_info()
_NC = _INFO.num_cores
_NS = _INFO.num_subcores
_L = _INFO.num_lanes
_NW = _NC * _NS
_B_PER_W = _BATCH // _NW
_CHUNK = 128
_NCHUNK = _B_PER_W // _CHUNK

_mesh = plsc.VectorSubcoreMesh(core_axis_name="c", subcore_axis_name="s")


@functools.partial(
    pl.kernel,
    mesh=_mesh,
    out_type=jax.ShapeDtypeStruct((_BATCH, _HIDDEN), jnp.float32),
    scratch_types=[
        pltpu.VMEM((_B_PER_W,), jnp.int32),
        pltpu.VMEM((_B_PER_W,), jnp.int32),
        pltpu.VMEM((_NCHUNK, _CHUNK), jnp.int32),
        pltpu.VMEM((_B_PER_W, _HIDDEN), jnp.float32),
        pltpu.SemaphoreType.DMA,
    ],
)
def _embed(labels_hbm, drop_hbm, table_hbm, out_hbm, lab_v, drop_v, idx_v, rows_v, sem):
    wid = lax.axis_index("s") * _NC + lax.axis_index("c")
    base = wid * _B_PER_W
    pltpu.sync_copy(labels_hbm.at[pl.ds(base, _B_PER_W)], lab_v)
    pltpu.sync_copy(drop_hbm.at[pl.ds(base, _B_PER_W)], drop_v)
    for i in range(_B_PER_W // _L):
        sl = pl.ds(i * _L, _L)
        lab = lab_v[sl]
        drp = drop_v[sl]
        idx_v[i // (_CHUNK // _L), pl.ds((i % (_CHUNK // _L)) * _L, _L)] = jnp.where(
            drp == 1, _NUM_CLASSES, lab
        )
    copies = []
    for j in range(_NCHUNK):
        copies.append(
            pltpu.async_copy(
                table_hbm.at[idx_v.at[j]],
                rows_v.at[pl.ds(j * _CHUNK, _CHUNK)],
                sem,
            )
        )
    for c in copies:
        c.wait()
    pltpu.sync_copy(rows_v, out_hbm.at[pl.ds(base, _B_PER_W)])


def kernel(labels, train, force_drop_ids, embedding_table):
    del train
    return _embed(labels, force_drop_ids, embedding_table)


# speedup vs baseline: 12.1899x; 12.1899x over previous
import functools

import jax
import jax.numpy as jnp
from jax import lax
from jax.experimental import pallas as pl
from jax.experimental.pallas import tpu as pltpu
from jax.experimental.pallas import tpu_sc as plsc

_NUM_CLASSES = 1000
_HIDDEN = 128
_BATCH = 16384
_NPAD = 1024

_INFO = plsc.get_sparse_core_info()
_NC = _INFO.num_cores
_NS = _INFO.num_subcores
_L = _INFO.num_lanes
_NW = _NC * _NS
_B_PER_W = _BATCH // _NW
_CHUNK = 128
_NCHUNK = _B_PER_W // _CHUNK

_mesh = plsc.VectorSubcoreMesh(core_axis_name="c", subcore_axis_name="s")


@functools.partial(
    pl.kernel,
    mesh=_mesh,
    out_type=jax.ShapeDtypeStruct((_BATCH, _HIDDEN), jnp.float32),
    scratch_types=[
        pltpu.VMEM((_B_PER_W,), jnp.int32),
        pltpu.VMEM((_B_PER_W,), jnp.int32),
        pltpu.VMEM((_NCHUNK, _CHUNK), jnp.int32),
        pltpu.VMEM((_B_PER_W, _HIDDEN), jnp.float32),
        pltpu.SemaphoreType.DMA,
    ],
)
def _embed(labels_hbm, drop_hbm, table_hbm, out_hbm, lab_v, drop_v, idx_v, rows_v, sem):
    wid = lax.axis_index("s") * _NC + lax.axis_index("c")
    base = wid * _B_PER_W
    pltpu.sync_copy(labels_hbm.at[pl.ds(base, _B_PER_W)], lab_v)
    pltpu.sync_copy(drop_hbm.at[pl.ds(base, _B_PER_W)], drop_v)
    lane = lax.iota(jnp.int32, _L)
    for i in range(_B_PER_W // _L):
        sl = pl.ds(i * _L, _L)
        lab = lab_v[sl]
        drp = drop_v[sl]
        spread = (base + i * _L + lane) & (_NPAD - 1)
        idx_v[i // (_CHUNK // _L), pl.ds((i % (_CHUNK // _L)) * _L, _L)] = jnp.where(
            drp == 1, _NUM_CLASSES + 1 + spread, lab
        )
    copies = []
    for j in range(_NCHUNK):
        copies.append(
            pltpu.async_copy(
                table_hbm.at[idx_v.at[j]],
                rows_v.at[pl.ds(j * _CHUNK, _CHUNK)],
                sem,
            )
        )
    for c in copies:
        c.wait()
    pltpu.sync_copy(rows_v, out_hbm.at[pl.ds(base, _B_PER_W)])


def kernel(labels, train, force_drop_ids, embedding_table):
    del train
    cfg_replicas = jnp.broadcast_to(embedding_table[_NUM_CLASSES], (_NPAD, _HIDDEN))
    table_ext = jnp.concatenate([embedding_table, cfg_replicas], axis=0)
    return _embed(labels, force_drop_ids, table_ext)


# speedup vs baseline: 12.6122x; 1.0346x over previous
import functools

import jax
import jax.numpy as jnp
from jax import lax
from jax.experimental import pallas as pl
from jax.experimental.pallas import tpu as pltpu
from jax.experimental.pallas import tpu_sc as plsc

_NUM_CLASSES = 1000
_HIDDEN = 128
_BATCH = 16384
_ROWS = _NUM_CLASSES + 1

_INFO = plsc.get_sparse_core_info()
_NC = _INFO.num_cores
_NS = _INFO.num_subcores
_L = _INFO.num_lanes
_NW = _NC * _NS
_B_PER_W = _BATCH // _NW
_CHUNK = 128
_NCHUNK = _B_PER_W // _CHUNK
_STAGE = 64

_mesh = plsc.VectorSubcoreMesh(core_axis_name="c", subcore_axis_name="s")


@functools.partial(
    pl.kernel,
    mesh=_mesh,
    out_type=jax.ShapeDtypeStruct((_BATCH, _HIDDEN), jnp.float32),
    scratch_types=[
        pltpu.VMEM_SHARED((_ROWS, _HIDDEN), jnp.float32),
        pltpu.VMEM((_B_PER_W,), jnp.int32),
        pltpu.VMEM((_B_PER_W,), jnp.int32),
        pltpu.VMEM((_NCHUNK, _CHUNK), jnp.int32),
        pltpu.VMEM((_B_PER_W, _HIDDEN), jnp.float32),
        pltpu.SemaphoreType.DMA,
    ],
)
def _embed(labels_hbm, drop_hbm, table_hbm, out_hbm, table_sp, lab_v, drop_v, idx_v, rows_v, sem):
    sid = lax.axis_index("s")
    wid = sid * _NC + lax.axis_index("c")
    base = wid * _B_PER_W
    @pl.when(sid < _NS - 1)
    def _stage_body():
        start = pl.multiple_of(sid * _STAGE, 8)
        pltpu.sync_copy(
            table_hbm.at[pl.ds(start, _STAGE)], table_sp.at[pl.ds(start, _STAGE)]
        )

    @pl.when(sid == _NS - 1)
    def _stage_tail():
        tail = (_NS - 1) * _STAGE
        pltpu.sync_copy(
            table_hbm.at[pl.ds(tail, _ROWS - tail)],
            table_sp.at[pl.ds(tail, _ROWS - tail)],
        )
    pltpu.sync_copy(labels_hbm.at[pl.ds(base, _B_PER_W)], lab_v)
    pltpu.sync_copy(drop_hbm.at[pl.ds(base, _B_PER_W)], drop_v)
    for i in range(_B_PER_W // _L):
        sl = pl.ds(i * _L, _L)
        lab = lab_v[sl]
        drp = drop_v[sl]
        idx_v[i // (_CHUNK // _L), pl.ds((i % (_CHUNK // _L)) * _L, _L)] = jnp.where(
            drp == 1, _NUM_CLASSES, lab
        )
    plsc.subcore_barrier()
    copies = []
    for j in range(_NCHUNK):
        copies.append(
            pltpu.async_copy(
                table_sp.at[idx_v.at[j]],
                rows_v.at[pl.ds(j * _CHUNK, _CHUNK)],
                sem,
            )
        )
    for c in copies:
        c.wait()
    pltpu.sync_copy(rows_v, out_hbm.at[pl.ds(base, _B_PER_W)])


def kernel(labels, train, force_drop_ids, embedding_table):
    del train
    return _embed(labels, force_drop_ids, embedding_table)


# speedup vs baseline: 13.6326x; 1.0809x over previous
import functools

import jax
import jax.numpy as jnp
from jax import lax
from jax.experimental import pallas as pl
from jax.experimental.pallas import tpu as pltpu
from jax.experimental.pallas import tpu_sc as plsc

_NUM_CLASSES = 1000
_HIDDEN = 128
_BATCH = 16384
_ROWS = _NUM_CLASSES + 1

_INFO = plsc.get_sparse_core_info()
_NC = _INFO.num_cores
_NS = _INFO.num_subcores
_L = _INFO.num_lanes
_NW = _NC * _NS
_B_PER_W = _BATCH // _NW
_CHUNK = 128
_NCHUNK = _B_PER_W // _CHUNK
_STAGE = 64

_mesh = plsc.VectorSubcoreMesh(core_axis_name="c", subcore_axis_name="s")


@functools.partial(
    pl.kernel,
    mesh=_mesh,
    out_type=jax.ShapeDtypeStruct((_BATCH, _HIDDEN), jnp.float32),
    scratch_types=[
        pltpu.VMEM_SHARED((_ROWS, _HIDDEN), jnp.float32),
        pltpu.VMEM((_B_PER_W,), jnp.int32),
        pltpu.VMEM((_B_PER_W,), jnp.int32),
        pltpu.VMEM((_B_PER_W,), jnp.int32),
        pltpu.VMEM((_B_PER_W, _HIDDEN), jnp.float32),
        pltpu.SemaphoreType.DMA,
        pltpu.SemaphoreType.DMA((_NCHUNK,)),
        pltpu.SemaphoreType.DMA,
    ],
)
def _embed(
    labels_hbm,
    drop_hbm,
    table_hbm,
    out_hbm,
    table_sp,
    lab_v,
    drop_v,
    idx_v,
    rows_v,
    sem_t,
    sem_g,
    sem_w,
):
    sid = lax.axis_index("s")
    wid = sid * _NC + lax.axis_index("c")
    base = wid * _B_PER_W
    @pl.when(sid < _NS - 1)
    def _stage_body():
        start = pl.multiple_of(sid * _STAGE, 8)
        pltpu.async_copy(
            table_hbm.at[pl.ds(start, _STAGE)], table_sp.at[pl.ds(start, _STAGE)], sem_t
        )

    tail = (_NS - 1) * _STAGE

    @pl.when(sid == _NS - 1)
    def _stage_tail():
        pltpu.async_copy(
            table_hbm.at[pl.ds(tail, _ROWS - tail)],
            table_sp.at[pl.ds(tail, _ROWS - tail)],
            sem_t,
        )

    pltpu.sync_copy(labels_hbm.at[pl.ds(base, _B_PER_W)], lab_v)
    pltpu.sync_copy(drop_hbm.at[pl.ds(base, _B_PER_W)], drop_v)

    def _adjust(i, carry):
        sl = pl.ds(pl.multiple_of(i * _L, _L), _L)
        idx_v[sl] = jnp.where(drop_v[sl] == 1, _NUM_CLASSES, lab_v[sl])
        return carry

    lax.fori_loop(0, _B_PER_W // _L, _adjust, 0)

    @pl.when(sid < _NS - 1)
    def _wait_body():
        start = pl.multiple_of(sid * _STAGE, 8)
        pltpu.make_async_copy(
            table_hbm.at[pl.ds(start, _STAGE)], table_sp.at[pl.ds(start, _STAGE)], sem_t
        ).wait()

    @pl.when(sid == _NS - 1)
    def _wait_tail():
        pltpu.make_async_copy(
            table_hbm.at[pl.ds(tail, _ROWS - tail)],
            table_sp.at[pl.ds(tail, _ROWS - tail)],
            sem_t,
        ).wait()

    plsc.subcore_barrier()

    gathers = []
    for j in range(_NCHUNK):
        gathers.append(
            pltpu.async_copy(
                table_sp.at[idx_v.at[pl.ds(j * _CHUNK, _CHUNK)]],
                rows_v.at[pl.ds(j * _CHUNK, _CHUNK)],
                sem_g.at[j],
            )
        )
    writebacks = []
    for j in range(_NCHUNK):
        gathers[j].wait()
        writebacks.append(
            pltpu.async_copy(
                rows_v.at[pl.ds(j * _CHUNK, _CHUNK)],
                out_hbm.at[pl.ds(base + j * _CHUNK, _CHUNK)],
                sem_w,
            )
        )
    for wb in writebacks:
        wb.wait()


def kernel(labels, train, force_drop_ids, embedding_table):
    del train
    return _embed(labels, force_drop_ids, embedding_table)


# speedup vs baseline: 13.7047x; 1.0053x over previous
import functools

import jax
import jax.numpy as jnp
from jax import lax
from jax.experimental import pallas as pl
from jax.experimental.pallas import tpu as pltpu
from jax.experimental.pallas import tpu_sc as plsc

_NUM_CLASSES = 1000
_HIDDEN = 128
_BATCH = 16384
_ROWS = _NUM_CLASSES + 1

_INFO = plsc.get_sparse_core_info()
_NC = _INFO.num_cores
_NS = _INFO.num_subcores
_L = _INFO.num_lanes
_NW = _NC * _NS
_B_PER_W = _BATCH // _NW
_CHUNK = 64
_NCHUNK = _B_PER_W // _CHUNK
_STAGE = 64

_mesh = plsc.VectorSubcoreMesh(core_axis_name="c", subcore_axis_name="s")


@functools.partial(
    pl.kernel,
    mesh=_mesh,
    out_type=jax.ShapeDtypeStruct((_BATCH, _HIDDEN), jnp.float32),
    scratch_types=[
        pltpu.VMEM_SHARED((_ROWS, _HIDDEN), jnp.float32),
        pltpu.VMEM((_B_PER_W,), jnp.int32),
        pltpu.VMEM((_B_PER_W,), jnp.int32),
        pltpu.VMEM((_B_PER_W,), jnp.int32),
        pltpu.VMEM((_B_PER_W, _HIDDEN), jnp.float32),
        pltpu.SemaphoreType.DMA,
        pltpu.SemaphoreType.DMA((_NCHUNK,)),
        pltpu.SemaphoreType.DMA,
    ],
)
def _embed(
    labels_hbm,
    drop_hbm,
    table_hbm,
    out_hbm,
    table_sp,
    lab_v,
    drop_v,
    idx_v,
    rows_v,
    sem_t,
    sem_g,
    sem_w,
):
    sid = lax.axis_index("s")
    wid = sid * _NC + lax.axis_index("c")
    base = wid * _B_PER_W
    @pl.when(sid < _NS - 1)
    def _stage_body():
        start = pl.multiple_of(sid * _STAGE, 8)
        pltpu.async_copy(
            table_hbm.at[pl.ds(start, _STAGE)], table_sp.at[pl.ds(start, _STAGE)], sem_t
        )

    tail = (_NS - 1) * _STAGE

    @pl.when(sid == _NS - 1)
    def _stage_tail():
        pltpu.async_copy(
            table_hbm.at[pl.ds(tail, _ROWS - tail)],
            table_sp.at[pl.ds(tail, _ROWS - tail)],
            sem_t,
        )

    pltpu.sync_copy(labels_hbm.at[pl.ds(base, _B_PER_W)], lab_v)
    pltpu.sync_copy(drop_hbm.at[pl.ds(base, _B_PER_W)], drop_v)

    def _adjust(i, carry):
        sl = pl.ds(pl.multiple_of(i * _L, _L), _L)
        idx_v[sl] = jnp.where(drop_v[sl] == 1, _NUM_CLASSES, lab_v[sl])
        return carry

    lax.fori_loop(0, _B_PER_W // _L, _adjust, 0)

    @pl.when(sid < _NS - 1)
    def _wait_body():
        start = pl.multiple_of(sid * _STAGE, 8)
        pltpu.make_async_copy(
            table_hbm.at[pl.ds(start, _STAGE)], table_sp.at[pl.ds(start, _STAGE)], sem_t
        ).wait()

    @pl.when(sid == _NS - 1)
    def _wait_tail():
        pltpu.make_async_copy(
            table_hbm.at[pl.ds(tail, _ROWS - tail)],
            table_sp.at[pl.ds(tail, _ROWS - tail)],
            sem_t,
        ).wait()

    plsc.subcore_barrier()

    gathers = []
    for j in range(_NCHUNK):
        gathers.append(
            pltpu.async_copy(
                table_sp.at[idx_v.at[pl.ds(j * _CHUNK, _CHUNK)]],
                rows_v.at[pl.ds(j * _CHUNK, _CHUNK)],
                sem_g.at[j],
            )
        )
    writebacks = []
    for j in range(_NCHUNK):
        gathers[j].wait()
        writebacks.append(
            pltpu.async_copy(
                rows_v.at[pl.ds(j * _CHUNK, _CHUNK)],
                out_hbm.at[pl.ds(base + j * _CHUNK, _CHUNK)],
                sem_w,
            )
        )
    for wb in writebacks:
        wb.wait()


def kernel(labels, train, force_drop_ids, embedding_table):
    del train
    return _embed(labels, force_drop_ids, embedding_table)
